# baseline (speedup 1.0000x reference)
import functools

import jax
import jax.numpy as jnp
from jax import lax
from jax.experimental import pallas as pl
from jax.experimental.pallas import tpu as pltpu
from jax.experimental.pallas import tpu_sc as plsc

BIN_SIZE = 20
MAX_DURATION = 1000
NB_BINS = MAX_DURATION // BIN_SIZE + 1

_NW = 32
_L = 16


def _sc_body(bin_hbm, spikes_hbm, out_hbm, spk_v, zbuf, idx_v, bin_v, semz, sems,
             *, rows_per_w, row_w, n):
    wid = lax.axis_index("s") * 2 + lax.axis_index("c")
    base = wid * rows_per_w
    nspk = rows_per_w * n

    pltpu.sync_copy(spikes_hbm.at[pl.ds(base * n, nspk)], spk_v)
    pltpu.sync_copy(bin_hbm, bin_v)
    binv = bin_v[...]
    iota = lax.iota(jnp.int32, _L)

    zeros = jnp.zeros((_L,), jnp.float32)

    def _memset(i, _):
        for u in range(8):
            zbuf[pl.ds(i * (8 * _L) + u * _L, _L)] = zeros
        return _

    lax.fori_loop(0, row_w // (8 * _L), _memset, None)

    zcopies = [
        pltpu.async_copy(
            zbuf, out_hbm.at[pl.ds((base + r) * row_w, row_w)], semz
        )
        for r in range(rows_per_w)
    ]

    def _build_idx(i, _):
        k = i * _L + iota
        r = lax.shift_right_logical(k, 10)
        nn = lax.bitwise_and(k, n - 1)
        pos = (base + r) * row_w + nn * NB_BINS + binv
        idx_v[i // 8, pl.ds((i % 8) * _L, _L)] = pos
        return _

    lax.fori_loop(0, nspk // _L, _build_idx, None)

    for cp in zcopies:
        cp.wait()

    scopies = [
        pltpu.async_copy(
            spk_v.at[pl.ds(j * 128, 128)], out_hbm.at[idx_v.at[j]], sems
        )
        for j in range(nspk // 128)
    ]
    for cp in scopies:
        cp.wait()


def kernel(spikes, time):
    B, N = spikes.shape
    row_w = N * NB_BINS
    bin_idx = jnp.full((_L,), jnp.asarray(time, jnp.int32) // BIN_SIZE, jnp.int32)
    rows_per_w = B // _NW

    mesh = plsc.VectorSubcoreMesh(core_axis_name="c", subcore_axis_name="s")
    sc_kernel = pl.kernel(
        functools.partial(_sc_body, rows_per_w=rows_per_w, row_w=row_w, n=N),
        mesh=mesh,
        out_type=jax.ShapeDtypeStruct((B * row_w,), spikes.dtype),
        scratch_types=[
            pltpu.VMEM((rows_per_w * N,), jnp.float32),
            pltpu.VMEM((row_w,), jnp.float32),
            pltpu.VMEM((128, 128), jnp.int32),
            pltpu.VMEM((_L,), jnp.int32),
            pltpu.SemaphoreType.DMA,
            pltpu.SemaphoreType.DMA,
        ],
    )
    out_flat = sc_kernel(bin_idx, spikes.reshape(-1))
    return out_flat.reshape(B, row_w)
